# baseline (speedup 1.0000x reference)
import functools

import jax
import jax.numpy as jnp
from jax.experimental import pallas as pl

_EPS = 1e-20
_ROWS = 8


def _gumbel_softmax_body(logits_ref, noise_ref, out_ref):
    x = logits_ref[...]
    u = noise_ref[...]
    w = 1.0 / (_EPS - jnp.log(u + _EPS))
    c = jnp.max(x, axis=-1, keepdims=True)
    e = jnp.exp(x - c) * w
    s = jnp.sum(e, axis=-1, keepdims=True)
    out_ref[...] = e * (1.0 / s)


@jax.jit
def kernel(logits, noise):
    batch, vocab = logits.shape
    rows = _ROWS
    grid = (batch // rows,)
    spec = pl.BlockSpec((rows, vocab), lambda i: (i, 0))
    return pl.pallas_call(
        _gumbel_softmax_body,
        grid=grid,
        in_specs=[spec, spec],
        out_specs=spec,
        out_shape=jax.ShapeDtypeStruct((batch, vocab), logits.dtype),
    )(logits, noise)
